# Initial kernel scaffold; baseline (speedup 1.0000x reference)
import jax
import jax.numpy as jnp
from jax.experimental import pallas as pl


def kernel(x, condition_ids, gamma_weight, beta_weight):
    raise NotImplementedError("write your pallas kernel here")


# speedup vs baseline: 1.0899x; 1.0899x over previous
import functools

import jax
import jax.numpy as jnp
from jax import lax
from jax.experimental import pallas as pl
from jax.experimental.pallas import tpu as pltpu
from jax.experimental.pallas import tpu_sc as plsc

N_CONDITIONS = 100000
HIDDEN = 128
BATCH = 16384

_INFO = plsc.get_sparse_core_info()
_NC = _INFO.num_cores
_NS = _INFO.num_subcores
_LANES = _INFO.num_lanes
_NW = _NC * _NS

_ROWS_PER_W = BATCH // _NW
_CHUNK = 128
_NCHUNKS = _ROWS_PER_W // _CHUNK
_VECS_PER_ROW = HIDDEN // _LANES


def _film_body(x_hbm, ids_hbm, gamma_hbm, beta_hbm, out_hbm,
               idx_v, g_v, b_v, x_v, gsem, bsem, xsem):
    wid = lax.axis_index("s") * _NC + lax.axis_index("c")
    base = wid * _ROWS_PER_W

    for j in range(_NCHUNKS):
        lo = base + j * _CHUNK
        pltpu.sync_copy(ids_hbm.at[pl.ds(lo, _CHUNK)], idx_v)
        gcp = pltpu.async_copy(gamma_hbm.at[idx_v], g_v, gsem)
        bcp = pltpu.async_copy(beta_hbm.at[idx_v], b_v, bsem)
        xcp = pltpu.async_copy(x_hbm.at[pl.ds(lo, _CHUNK)], x_v, xsem)
        gcp.wait()
        bcp.wait()
        xcp.wait()

        def row(i):
            for v in range(_VECS_PER_ROW):
                sl = pl.ds(v * _LANES, _LANES)
                g_v[i, sl] = g_v[i, sl] * x_v[i, sl] + b_v[i, sl]

        plsc.parallel_loop(0, _CHUNK, 1, unroll=4)(row)
        pltpu.sync_copy(g_v, out_hbm.at[pl.ds(lo, _CHUNK)])


@jax.jit
def kernel(x, condition_ids, gamma_weight, beta_weight):
    ids32 = condition_ids.astype(jnp.int32)
    film = pl.kernel(
        _film_body,
        out_type=jax.ShapeDtypeStruct((BATCH, HIDDEN), jnp.float32),
        mesh=plsc.VectorSubcoreMesh(core_axis_name="c", subcore_axis_name="s"),
        scratch_types=[
            pltpu.VMEM((_CHUNK,), jnp.int32),
            pltpu.VMEM((_CHUNK, HIDDEN), jnp.float32),
            pltpu.VMEM((_CHUNK, HIDDEN), jnp.float32),
            pltpu.VMEM((_CHUNK, HIDDEN), jnp.float32),
            pltpu.SemaphoreType.DMA,
            pltpu.SemaphoreType.DMA,
            pltpu.SemaphoreType.DMA,
        ],
    )
    return film(x, ids32, gamma_weight, beta_weight)


# speedup vs baseline: 1.2738x; 1.1687x over previous
import jax
import jax.numpy as jnp
from jax import lax
from jax.experimental import pallas as pl
from jax.experimental.pallas import tpu as pltpu
from jax.experimental.pallas import tpu_sc as plsc

N_CONDITIONS = 100000
HIDDEN = 128
BATCH = 16384

_INFO = plsc.get_sparse_core_info()
_NC = _INFO.num_cores
_NS = _INFO.num_subcores
_LANES = _INFO.num_lanes
_NW = _NC * _NS

_ROWS_PER_W = BATCH // _NW
_CHUNK = 128
_NCHUNKS = _ROWS_PER_W // _CHUNK
_VECS_PER_ROW = HIDDEN // _LANES


def _film_body(x_hbm, ids_hbm, gamma_hbm, beta_hbm, out_hbm,
               idx_v, g0, g1, b0, b1, x0, x1,
               gs0, gs1, bs0, bs1, xs0, xs1, os0, os1):
    g = (g0, g1)
    b = (b0, b1)
    x = (x0, x1)
    gs = (gs0, gs1)
    bs = (bs0, bs1)
    xs = (xs0, xs1)
    osem = (os0, os1)

    wid = lax.axis_index("s") * _NC + lax.axis_index("c")
    base = wid * _ROWS_PER_W

    for j in range(_NCHUNKS):
        pltpu.sync_copy(ids_hbm.at[pl.ds(base + j * _CHUNK, _CHUNK)],
                        idx_v.at[j])

    def issue(j):
        k = j % 2
        lo = base + j * _CHUNK
        return (
            pltpu.async_copy(gamma_hbm.at[idx_v.at[j]], g[k], gs[k]),
            pltpu.async_copy(beta_hbm.at[idx_v.at[j]], b[k], bs[k]),
            pltpu.async_copy(x_hbm.at[pl.ds(lo, _CHUNK)], x[k], xs[k]),
        )

    pending = issue(0)
    store_pending = [None, None]
    for j in range(_NCHUNKS):
        k = j % 2
        nk = (j + 1) % 2
        if j + 1 < _NCHUNKS:
            if store_pending[nk] is not None:
                store_pending[nk].wait()
                store_pending[nk] = None
            nxt = issue(j + 1)
        for c in pending:
            c.wait()

        gk, bk, xk = g[k], b[k], x[k]

        def row(i):
            for v in range(_VECS_PER_ROW):
                sl = pl.ds(v * _LANES, _LANES)
                gk[i, sl] = gk[i, sl] * xk[i, sl] + bk[i, sl]

        plsc.parallel_loop(0, _CHUNK, 1, unroll=4)(row)
        store_pending[k] = pltpu.async_copy(
            gk, out_hbm.at[pl.ds(base + j * _CHUNK, _CHUNK)], osem[k])
        if j + 1 < _NCHUNKS:
            pending = nxt
    for sp in store_pending:
        if sp is not None:
            sp.wait()


@jax.jit
def kernel(x, condition_ids, gamma_weight, beta_weight):
    ids32 = condition_ids.astype(jnp.int32)
    film = pl.kernel(
        _film_body,
        out_type=jax.ShapeDtypeStruct((BATCH, HIDDEN), jnp.float32),
        mesh=plsc.VectorSubcoreMesh(core_axis_name="c", subcore_axis_name="s"),
        scratch_types=[
            pltpu.VMEM((_NCHUNKS, _CHUNK), jnp.int32),
            pltpu.VMEM((_CHUNK, HIDDEN), jnp.float32),
            pltpu.VMEM((_CHUNK, HIDDEN), jnp.float32),
            pltpu.VMEM((_CHUNK, HIDDEN), jnp.float32),
            pltpu.VMEM((_CHUNK, HIDDEN), jnp.float32),
            pltpu.VMEM((_CHUNK, HIDDEN), jnp.float32),
            pltpu.VMEM((_CHUNK, HIDDEN), jnp.float32),
            pltpu.SemaphoreType.DMA,
            pltpu.SemaphoreType.DMA,
            pltpu.SemaphoreType.DMA,
            pltpu.SemaphoreType.DMA,
            pltpu.SemaphoreType.DMA,
            pltpu.SemaphoreType.DMA,
            pltpu.SemaphoreType.DMA,
            pltpu.SemaphoreType.DMA,
        ],
    )
    return film(x, ids32, gamma_weight, beta_weight)
